# baseline (speedup 1.0000x reference)
import functools

import jax
import jax.numpy as jnp
from jax import lax
from jax.experimental import pallas as pl
from jax.experimental.pallas import tpu as pltpu
from jax.experimental.pallas import tpu_sc as plsc

_N_E = 8192
_E_DIM = 32
_BETA = 0.25
_HW = 1024
_CB_CHUNK = 2048
_SLICE = 128
_CG = 256


def _argmin_body(z_ref, cb_ref, idx_ref, loss_ref):
    zb = z_ref[0]
    znorm = jnp.sum(zb * zb, axis=0, keepdims=True)
    zm2 = zb * (-2.0)
    cn_full = jnp.sum(cb_ref[...] * cb_ref[...], axis=1,
                      keepdims=True)

    part = jnp.float32(0.0)
    for g in range(_HW // _CG):
        zg = zm2[:, g * _CG:(g + 1) * _CG]
        zn = znorm[:, g * _CG:(g + 1) * _CG]
        mv = jnp.full((_SLICE, _CG), jnp.inf, dtype=jnp.float32)
        mbl = jnp.zeros((_SLICE, _CG), dtype=jnp.int32)
        for c in range(_N_E // _CB_CHUNK):
            cb = cb_ref[pl.ds(c * _CB_CHUNK, _CB_CHUNK), :]
            cn = cn_full[c * _CB_CHUNK:(c + 1) * _CB_CHUNK, :]
            dot = lax.dot_general(cb, zg, (((1,), (0,)), ((), ())),
                                  preferred_element_type=jnp.float32)
            s = (zn + cn) + dot
            for b in range(_CB_CHUNK // _SLICE):
                sb = s[b * _SLICE:(b + 1) * _SLICE, :]
                blk = c * (_CB_CHUNK // _SLICE) + b
                upd = sb < mv
                mv = jnp.minimum(sb, mv)
                mbl = jnp.where(upd, blk, mbl)

        jfull = mbl * _SLICE + lax.broadcasted_iota(
            jnp.int32, (_SLICE, _CG), 0)
        m = jnp.min(mv, axis=0, keepdims=True)
        mi = jnp.min(jnp.where(mv == m, jfull, _N_E), axis=0, keepdims=True)
        idx_ref[0, 0, pl.ds(g * _CG, _CG)] = mi[0]
        part += jnp.sum(m)
    i = pl.program_id(0)

    @pl.when(i == 0)
    def _():
        loss_ref[0, 0] = part

    @pl.when(i > 0)
    def _():
        loss_ref[0, 0] += part

    @pl.when(i == pl.num_programs(0) - 1)
    def _():
        loss_ref[0, 0] *= (1.0 + _BETA) / (4 * _HW * _E_DIM)


def _tc_argmin(z3, codebook):
    nb = z3.shape[0]
    return pl.pallas_call(
        _argmin_body,
        grid=(nb,),
        in_specs=[
            pl.BlockSpec((1, _E_DIM, _HW), lambda i: (i, 0, 0)),
            pl.BlockSpec((_N_E, _E_DIM), lambda i: (0, 0)),
        ],
        out_specs=[
            pl.BlockSpec((1, 1, _HW), lambda i: (i, 0, 0)),
            pl.BlockSpec(memory_space=pltpu.SMEM),
        ],
        out_shape=[
            jax.ShapeDtypeStruct((nb, 1, _HW), jnp.int32),
            jax.ShapeDtypeStruct((1, 1), jnp.float32),
        ],
    )(z3, codebook)


def _sc_gather(codebook, idx):
    b = idx.shape[0]
    info = plsc.get_sparse_core_info()
    nw = info.num_cores * info.num_subcores
    bpw = b // nw
    mesh = plsc.VectorSubcoreMesh(core_axis_name="c", subcore_axis_name="s")

    @functools.partial(
        pl.kernel,
        out_type=jax.ShapeDtypeStruct((b, _E_DIM), jnp.float32),
        mesh=mesh,
        scratch_types=[
            pltpu.VMEM((bpw,), jnp.int32),
            pltpu.VMEM((bpw, _E_DIM), jnp.float32),
            pltpu.SemaphoreType.DMA,
        ],
        compiler_params=pltpu.CompilerParams(use_tc_tiling_on_sc=False,
                                             skip_device_barrier=True),
    )
    def gather_k(table_hbm, idx_hbm, out_hbm, idx_v, rows_v, sem):
        wid = lax.axis_index("s") * info.num_cores + lax.axis_index("c")
        base = wid * bpw
        pltpu.sync_copy(idx_hbm.at[pl.ds(base, bpw)], idx_v)
        pltpu.async_copy(table_hbm.at[idx_v], rows_v, sem).wait()
        pltpu.sync_copy(rows_v, out_hbm.at[pl.ds(base, bpw)])

    return gather_k(codebook, idx)


def kernel(z, codebook):
    b, c, h, w = z.shape
    z3 = z.reshape(b, c, h * w)

    idx3d, loss2d = _tc_argmin(z3, codebook)
    idx = idx3d.reshape(-1)

    z_q = _sc_gather(codebook, idx)

    loss = loss2d.reshape(())
    z_q_out = jnp.transpose(z_q.reshape(b, h, w, c), (0, 3, 1, 2))
    indices_out = idx3d.reshape(b, 1, h, w)
    return z_q_out, loss, indices_out


# speedup vs baseline: 1.0226x; 1.0226x over previous
import functools

import jax
import jax.numpy as jnp
from jax import lax
from jax.experimental import pallas as pl
from jax.experimental.pallas import tpu as pltpu
from jax.experimental.pallas import tpu_sc as plsc

_N_E = 8192
_E_DIM = 32
_BETA = 0.25
_HW = 1024
_CB_CHUNK = 2048
_SLICE = 128
_CG = 256


def _argmin_body(z_ref, cb_ref, idx_ref, loss_ref):
    zb = z_ref[0].reshape(_E_DIM, _HW)
    znorm = jnp.sum(zb * zb, axis=0, keepdims=True)
    zm2 = zb * (-2.0)
    cn_full = jnp.sum(cb_ref[...] * cb_ref[...], axis=1,
                      keepdims=True)

    part = jnp.float32(0.0)
    for g in range(_HW // _CG):
        zg = zm2[:, g * _CG:(g + 1) * _CG]
        zn = znorm[:, g * _CG:(g + 1) * _CG]
        mv = jnp.full((_SLICE, _CG), jnp.inf, dtype=jnp.float32)
        mbl = jnp.zeros((_SLICE, _CG), dtype=jnp.int32)
        for c in range(_N_E // _CB_CHUNK):
            cb = cb_ref[pl.ds(c * _CB_CHUNK, _CB_CHUNK), :]
            cn = cn_full[c * _CB_CHUNK:(c + 1) * _CB_CHUNK, :]
            dot = lax.dot_general(cb, zg, (((1,), (0,)), ((), ())),
                                  preferred_element_type=jnp.float32)
            s = (zn + cn) + dot
            for b in range(_CB_CHUNK // _SLICE):
                sb = s[b * _SLICE:(b + 1) * _SLICE, :]
                blk = c * (_CB_CHUNK // _SLICE) + b
                upd = sb < mv
                mv = jnp.minimum(sb, mv)
                mbl = jnp.where(upd, blk, mbl)

        jfull = mbl * _SLICE + lax.broadcasted_iota(
            jnp.int32, (_SLICE, _CG), 0)
        m = jnp.min(mv, axis=0, keepdims=True)
        mi = jnp.min(jnp.where(mv == m, jfull, _N_E), axis=0, keepdims=True)
        idx_ref[0, 0, pl.ds(g * _CG, _CG)] = mi[0]
        part += jnp.sum(m)
    i = pl.program_id(0)

    @pl.when(i == 0)
    def _():
        loss_ref[0, 0] = part

    @pl.when(i > 0)
    def _():
        loss_ref[0, 0] += part

    @pl.when(i == pl.num_programs(0) - 1)
    def _():
        loss_ref[0, 0] *= (1.0 + _BETA) / (4 * _HW * _E_DIM)


def _tc_argmin(z3, codebook):
    nb = z3.shape[0]
    return pl.pallas_call(
        _argmin_body,
        grid=(nb,),
        in_specs=[
            pl.BlockSpec((1, _E_DIM, 32, 32), lambda i: (i, 0, 0, 0)),
            pl.BlockSpec((_N_E, _E_DIM), lambda i: (0, 0)),
        ],
        out_specs=[
            pl.BlockSpec((1, 1, _HW), lambda i: (i, 0, 0)),
            pl.BlockSpec(memory_space=pltpu.SMEM),
        ],
        out_shape=[
            jax.ShapeDtypeStruct((nb, 1, _HW), jnp.int32),
            jax.ShapeDtypeStruct((1, 1), jnp.float32),
        ],
    )(z3, codebook)


def _sc_gather(codebook, idx):
    b = idx.shape[0]
    info = plsc.get_sparse_core_info()
    nw = info.num_cores * info.num_subcores
    bpw = b // nw
    mesh = plsc.VectorSubcoreMesh(core_axis_name="c", subcore_axis_name="s")

    @functools.partial(
        pl.kernel,
        out_type=jax.ShapeDtypeStruct((b, _E_DIM), jnp.float32),
        mesh=mesh,
        scratch_types=[
            pltpu.VMEM((bpw,), jnp.int32),
            pltpu.VMEM((bpw, _E_DIM), jnp.float32),
            pltpu.SemaphoreType.DMA,
        ],
        compiler_params=pltpu.CompilerParams(use_tc_tiling_on_sc=False),
    )
    def gather_k(table_hbm, idx_hbm, out_hbm, idx_v, rows_v, sem):
        wid = lax.axis_index("s") * info.num_cores + lax.axis_index("c")
        base = wid * bpw
        pltpu.sync_copy(idx_hbm.at[pl.ds(base, bpw)], idx_v)
        pltpu.async_copy(table_hbm.at[idx_v], rows_v, sem).wait()
        pltpu.sync_copy(rows_v, out_hbm.at[pl.ds(base, bpw)])

    return gather_k(codebook, idx)


def kernel(z, codebook):
    b, c, h, w = z.shape

    idx3d, loss2d = _tc_argmin(z, codebook)
    idx = idx3d.reshape(-1)

    z_q = _sc_gather(codebook, idx)

    loss = loss2d.reshape(())
    z_q_out = jnp.transpose(z_q.reshape(b, h, w, c), (0, 3, 1, 2))
    indices_out = idx3d.reshape(b, 1, h, w)
    return z_q_out, loss, indices_out
